# baseline (speedup 1.0000x reference)
import functools

import jax
import jax.numpy as jnp
import numpy as np
from jax import lax
from jax.experimental import pallas as pl
from jax.experimental.pallas import tpu as pltpu
from jax.experimental.pallas import tpu_sc as plsc

N_USER = 25000
N_ITEM = 25000
N = N_USER + N_ITEM
D = 64
DH = D // 2
NNZ = 800000
LANES = 16

NC = 2
NS = 16
NW = NC * NS

NP = 50176
ZPT = NP // NS

NNZ_P = 819200
EPT = NNZ_P // NS
CHUNK = 128
GC = 2
GROUP = CHUNK * GC
NGROUPS = EPT // GROUP

F = 4
NPF = NP // F
DB = 448
TC_GRID = NPF // DB

GB = 3 * 1024
GPW = GB // NW

_PERM = np.array([(m // 2) if m % 2 == 0 else 16 + m // 2
                  for m in range(DH)])
_PSLO = np.zeros((F * D, F * DH), np.float32)
_PSHI = np.zeros((F * D, F * DH), np.float32)
for _k in range(F):
    for _m in range(DH):
        _PSLO[_k * D + _PERM[_m], _k * DH + _m] = 1.0
        _PSHI[_k * D + DH + _PERM[_m], _k * DH + _m] = 1.0


def _pack_bf16(x):
    xp = x[:, _PERM].astype(jnp.bfloat16)
    return lax.bitcast_convert_type(xp.reshape(NP, LANES, 2), jnp.int32)


def _spmm_body(ego_lo, ego_hi, adj3, val2, zrows, out_lo, out_hi,
               r0, r1, r2, r3, c0, c1, c2, c3, v0, v1, v2, v3, gb0, gb1,
               mb0, mb1, semg0, semg1, sems0, sems1, semm0, semm1, acc):
    core = lax.axis_index("c")
    sub = lax.axis_index("s")
    rows = [r0, r1, r2, r3]
    cols = [c0, c1, c2, c3]
    vals = [v0, v1, v2, v3]
    gbufs = [gb0, gb1]
    mbufs = [mb0, mb1]
    semg = [semg0, semg1]
    sems = [sems0, sems1]
    semm = [semm0, semm1]

    pltpu.sync_copy(zrows, acc.at[pl.ds(sub * ZPT, ZPT)])
    plsc.subcore_barrier()

    cbase = sub * (EPT // CHUNK)

    def meta_copies(i, slot, sem, make):
        f = pltpu.make_async_copy if make else pltpu.async_copy
        sl = pl.ds(cbase + i * GC, GC)
        return [f(adj3.at[0, sl], rows[slot], sem),
                f(adj3.at[1, sl], cols[slot], sem),
                f(val2.at[sl], vals[slot], sem)]

    def issue_gathers(ego, slot, p):
        return [pltpu.async_copy(
            ego.at[cols[slot].at[c]],
            gbufs[p].at[pl.ds(c * CHUNK, CHUNK)], semg[p])
            for c in range(GC)]

    def scale(slot, p):
        gbuf = gbufs[p]
        mbuf = mbufs[p]
        vv_ref = vals[slot]
        himask = jnp.int32(-65536)

        def body(i, _):
            c = i // (CHUNK // LANES)
            o = (i % (CHUNK // LANES)) * LANES
            vv = vv_ref[c, pl.ds(o, LANES)]
            for k in range(LANES):
                e = i * LANES + k
                vs = jnp.full((LANES,), vv[k], jnp.float32)
                w = gbuf[e, pl.ds(0, LANES)]
                lo = lax.bitcast_convert_type(w << 16, jnp.float32)
                hi = lax.bitcast_convert_type(w & himask, jnp.float32)
                mbuf[e, pl.ds(0, LANES)] = lo * vs
                mbuf[e, pl.ds(LANES, LANES)] = hi * vs
            return 0
        lax.fori_loop(0, GROUP // LANES, body, 0, unroll=False)

    def run(ego, out):
        for cp in meta_copies(0, 0, semm[0], False):
            cp.wait()
        meta_copies(1, 1, semm[1], False)
        issue_gathers(ego, 0, 0)

        def outer(g4, _):
            for j in range(4):
                i = g4 * 4 + j
                p, q = j % 2, 1 - (j % 2)
                for c in range(GC):
                    pltpu.make_async_copy(
                        ego.at[cols[j].at[c]],
                        gbufs[p].at[pl.ds(c * CHUNK, CHUNK)],
                        semg[p]).wait()
                @pl.when(i + 2 < NGROUPS)
                def _():
                    meta_copies(i + 2, (j + 2) % 4, semm[p], False)
                @pl.when(i + 1 < NGROUPS)
                def _():
                    for cp in meta_copies(i + 1, (j + 1) % 4, semm[q], True):
                        cp.wait()

                @pl.when(i >= 1)
                def _():
                    for c in range(GC):
                        pltpu.make_async_copy(
                            mbufs[q].at[pl.ds(c * CHUNK, CHUNK)],
                            acc.at[rows[(j + 3) % 4].at[c]],
                            sems[q]).wait()

                @pl.when(i + 1 < NGROUPS)
                def _():
                    issue_gathers(ego, (j + 1) % 4, q)

                scale(j, p)
                for c in range(GC):
                    pltpu.async_copy(
                        mbufs[p].at[pl.ds(c * CHUNK, CHUNK)],
                        acc.at[rows[j].at[c]], sems[p], add=True)
            return 0

        lax.fori_loop(0, NGROUPS // 4, outer, 0, unroll=False)

        for c in range(GC):
            pltpu.make_async_copy(
                mbufs[1].at[pl.ds(c * CHUNK, CHUNK)],
                acc.at[rows[3].at[c]], sems[1]).wait()

        plsc.subcore_barrier()
        pltpu.sync_copy(acc.at[pl.ds(sub * ZPT, ZPT)],
                        out.at[pl.ds(sub * ZPT, ZPT)])

    @pl.when(core == 0)
    def _():
        run(ego_lo, out_lo)

    @pl.when(core == 1)
    def _():
        run(ego_hi, out_hi)


@jax.jit
def _spmm(ego_lo, ego_hi, adj3, val2):
    zrows = jnp.zeros((ZPT, DH), jnp.float32)
    mesh = plsc.VectorSubcoreMesh(core_axis_name="c", subcore_axis_name="s")
    half = jax.ShapeDtypeStruct((NP, DH), jnp.float32)
    idxbuf = pltpu.VMEM((GC, CHUNK), jnp.int32)
    valbuf = pltpu.VMEM((GC, CHUNK), jnp.float32)
    return pl.kernel(
        _spmm_body,
        out_type=(half, half),
        mesh=mesh,
        scratch_types=[
            idxbuf, idxbuf, idxbuf, idxbuf,
            idxbuf, idxbuf, idxbuf, idxbuf,
            valbuf, valbuf, valbuf, valbuf,
            pltpu.VMEM((GROUP, LANES), jnp.int32),
            pltpu.VMEM((GROUP, LANES), jnp.int32),
            pltpu.VMEM((GROUP, DH), jnp.float32),
            pltpu.VMEM((GROUP, DH), jnp.float32),
            pltpu.SemaphoreType.DMA,
            pltpu.SemaphoreType.DMA,
            pltpu.SemaphoreType.DMA,
            pltpu.SemaphoreType.DMA,
            pltpu.SemaphoreType.DMA,
            pltpu.SemaphoreType.DMA,
            pltpu.VMEM_SHARED((NP, DH), jnp.float32),
        ],
        compiler_params=pltpu.CompilerParams(use_tc_tiling_on_sc=False),
    )(ego_lo, ego_hi, adj3, val2, zrows)


def _dense_body(elo, ehi, slo, shi, wgl, wgh, bg4, wbl, wbh, bb4, ones4,
                pslo, pshi, new_lo, new_hi, norm, blo, bhi):
    e_lo, e_hi = elo[...], ehi[...]
    s_lo, s_hi = slo[...], shi[...]
    x = (jnp.dot(s_lo, wgl[...], preferred_element_type=jnp.float32)
         + jnp.dot(s_hi, wgh[...], preferred_element_type=jnp.float32)
         + jnp.dot(e_lo * s_lo, wbl[...], preferred_element_type=jnp.float32)
         + jnp.dot(e_hi * s_hi, wbh[...], preferred_element_type=jnp.float32)
         + bg4[...] + bb4[...])
    h = jnp.where(x >= 0, x, 0.2 * x)
    n2 = jnp.dot(h * h, ones4[...], preferred_element_type=jnp.float32)
    new_lo[...] = jnp.concatenate(
        [h[:, k * D:k * D + DH] for k in range(F)], axis=1)
    new_hi[...] = jnp.concatenate(
        [h[:, k * D + DH:(k + 1) * D] for k in range(F)], axis=1)
    blo[...] = jnp.dot(h, pslo[...],
                       preferred_element_type=jnp.float32).astype(jnp.bfloat16)
    bhi[...] = jnp.dot(h, pshi[...],
                       preferred_element_type=jnp.float32).astype(jnp.bfloat16)
    norm[...] = h / jnp.maximum(jnp.sqrt(n2), 1e-12)


@jax.jit
def _dense(elo, ehi, slo, shi, wgl, wgh, bg4, wbl, wbh, bb4, ones4,
           pslo, pshi):
    blk = pl.BlockSpec((DB, 128), lambda i: (i, 0))
    blk_w = pl.BlockSpec((DB, F * D), lambda i: (i, 0))
    wspec = pl.BlockSpec((128, F * D), lambda i: (0, 0))
    ospec = pl.BlockSpec((F * D, F * D), lambda i: (0, 0))
    bspec = pl.BlockSpec((1, F * D), lambda i: (0, 0))
    pspec = pl.BlockSpec((F * D, 128), lambda i: (0, 0))
    fold = jax.ShapeDtypeStruct((NPF, 128), jnp.float32)
    foldb = jax.ShapeDtypeStruct((NPF, 128), jnp.bfloat16)
    return pl.pallas_call(
        _dense_body,
        grid=(TC_GRID,),
        in_specs=[blk, blk, blk, blk,
                  wspec, wspec, bspec, wspec, wspec, bspec, ospec,
                  pspec, pspec],
        out_specs=[blk, blk, blk_w, blk, blk],
        out_shape=[fold, fold,
                   jax.ShapeDtypeStruct((NPF, F * D), jnp.float32),
                   foldb, foldb],
    )(elo, ehi, slo, shi, wgl, wgh, bg4, wbl, wbh, bb4, ones4, pslo, pshi)


def _lookup_body(s0, s1, s2, s3, idx, o0, o1, o2, o3, idxv, rbuf, sem):
    core = lax.axis_index("c")
    sub = lax.axis_index("s")
    wid = sub * NC + core
    base = wid * GPW
    pltpu.sync_copy(idx.at[pl.ds(base, GPW)], idxv)
    for slab, o in ((s0, o0), (s1, o1), (s2, o2), (s3, o3)):
        pltpu.async_copy(slab.at[idxv], rbuf, sem).wait()
        pltpu.sync_copy(rbuf, o.at[pl.ds(base, GPW)])


@jax.jit
def _lookup(s0, s1, s2, s3, idx):
    mesh = plsc.VectorSubcoreMesh(core_axis_name="c", subcore_axis_name="s")
    out = jax.ShapeDtypeStruct((GB, D), jnp.float32)
    return pl.kernel(
        _lookup_body,
        out_type=(out, out, out, out),
        mesh=mesh,
        scratch_types=[
            pltpu.VMEM((GPW,), jnp.int32),
            pltpu.VMEM((GPW, D), jnp.float32),
            pltpu.SemaphoreType.DMA,
        ],
        compiler_params=pltpu.CompilerParams(use_tc_tiling_on_sc=False),
    )(s0, s1, s2, s3, idx)


def kernel(user_emb, item_emb,
           W_gc_0, b_gc_0, W_bi_0, b_bi_0,
           W_gc_1, b_gc_1, W_bi_1, b_bi_1,
           W_gc_2, b_gc_2, W_bi_2, b_bi_2,
           adj_vals, adj_idx, users, pos_items, neg_items):
    Ws = [(W_gc_0, b_gc_0, W_bi_0, b_bi_0),
          (W_gc_1, b_gc_1, W_bi_1, b_bi_1),
          (W_gc_2, b_gc_2, W_bi_2, b_bi_2)]

    ego0 = jnp.zeros((NP, D), jnp.float32)
    ego0 = ego0.at[:N_USER].set(user_emb).at[N_USER:N].set(item_emb)

    adj3 = jnp.pad(adj_idx.astype(jnp.int32),
                   ((0, 0), (0, NNZ_P - NNZ))).reshape(2, -1, CHUNK)
    val2 = jnp.pad(adj_vals.astype(jnp.float32),
                   (0, NNZ_P - NNZ)).reshape(-1, CHUNK)

    eye4 = jnp.eye(F, dtype=jnp.float32)
    ones4 = jnp.kron(eye4, jnp.ones((D, D), jnp.float32))
    pslo = jnp.asarray(_PSLO)
    pshi = jnp.asarray(_PSHI)

    slabs = [ego0]
    glo = _pack_bf16(ego0[:, :DH])
    ghi = _pack_bf16(ego0[:, DH:])
    ef_lo = ego0[:, :DH].reshape(NPF, 128)
    ef_hi = ego0[:, DH:].reshape(NPF, 128)
    for (wgc, bgc, wbi, bbi) in Ws:
        slo, shi = _spmm(glo, ghi, adj3, val2)
        wgl = jnp.kron(eye4, wgc[:DH])
        wgh = jnp.kron(eye4, wgc[DH:])
        wbl = jnp.kron(eye4, wbi[:DH])
        wbh = jnp.kron(eye4, wbi[DH:])
        bg4 = jnp.tile(bgc, (1, F))
        bb4 = jnp.tile(bbi, (1, F))
        ef_lo, ef_hi, nrm, blo, bhi = _dense(
            ef_lo, ef_hi, slo.reshape(NPF, 128), shi.reshape(NPF, 128),
            wgl, wgh, bg4, wbl, wbh, bb4, ones4, pslo, pshi)
        glo = lax.bitcast_convert_type(
            blo.reshape(NP, LANES, 2), jnp.int32)
        ghi = lax.bitcast_convert_type(
            bhi.reshape(NP, LANES, 2), jnp.int32)
        slabs.append(nrm.reshape(NP, D))

    idx = jnp.concatenate([users.astype(jnp.int32),
                           pos_items.astype(jnp.int32) + N_USER,
                           neg_items.astype(jnp.int32) + N_USER])
    g0, g1, g2, g3 = _lookup(slabs[0], slabs[1], slabs[2], slabs[3], idx)
    all_e = jnp.concatenate([g0, g1, g2, g3], axis=1)
    B = users.shape[0]
    return (all_e[:B], all_e[B:2 * B], all_e[2 * B:])


# speedup vs baseline: 5.7646x; 5.7646x over previous
import functools

import jax
import jax.numpy as jnp
import numpy as np
from jax import lax
from jax.experimental import pallas as pl
from jax.experimental.pallas import tpu as pltpu
from jax.experimental.pallas import tpu_sc as plsc

N_USER = 25000
N_ITEM = 25000
N = N_USER + N_ITEM
D = 64
DH = D // 2
NNZ = 800000
LANES = 16

NC = 2
NS = 16
NW = NC * NS

NP = 50176
ZPT = NP // NS

NNZ_P = 819200
EPT = NNZ_P // NS
CHUNK = 128
GC = 2
GROUP = CHUNK * GC
NGROUPS = EPT // GROUP

F = 4
NPF = NP // F
DB = 448
TC_GRID = NPF // DB

GB = 3 * 1024
GPW = GB // NW

def _pack_half(x):
    xr = lax.bitcast_convert_type(
        x.astype(jnp.bfloat16).astype(jnp.float32), jnp.int32)
    return (((xr[:, :LANES] >> 16) & jnp.int32(65535))
            | (xr[:, LANES:] & jnp.int32(-65536)))


def _pack_fold(h, off):
    return jnp.concatenate(
        [_pack_half(h[:, k * D + off:k * D + off + DH]) for k in range(F)],
        axis=1)


def _spmm_body(ego_lo, ego_hi, adj3, val2, zrows, out_lo, out_hi,
               r0, r1, r2, r3, c0, c1, c2, c3, v0, v1, v2, v3, gb0, gb1,
               mb0, mb1, semg0, semg1, sems0, sems1, semm0, semm1, acc):
    core = lax.axis_index("c")
    sub = lax.axis_index("s")
    rows = [r0, r1, r2, r3]
    cols = [c0, c1, c2, c3]
    vals = [v0, v1, v2, v3]
    gbufs = [gb0, gb1]
    mbufs = [mb0, mb1]
    semg = [semg0, semg1]
    sems = [sems0, sems1]
    semm = [semm0, semm1]

    pltpu.sync_copy(zrows, acc.at[pl.ds(sub * ZPT, ZPT)])
    plsc.subcore_barrier()

    cbase = sub * (EPT // CHUNK)

    def meta_copies(i, slot, sem, make):
        f = pltpu.make_async_copy if make else pltpu.async_copy
        sl = pl.ds(cbase + i * GC, GC)
        return [f(adj3.at[0, sl], rows[slot], sem),
                f(adj3.at[1, sl], cols[slot], sem),
                f(val2.at[sl], vals[slot], sem)]

    def issue_gathers(ego, slot, p):
        return [pltpu.async_copy(
            ego.at[cols[slot].at[c]],
            gbufs[p].at[pl.ds(c * CHUNK, CHUNK)], semg[p])
            for c in range(GC)]

    def scale(slot, p):
        gbuf = gbufs[p]
        mbuf = mbufs[p]
        vv_ref = vals[slot]
        himask = jnp.int32(-65536)

        def body(i, _):
            c = i // (CHUNK // LANES)
            o = (i % (CHUNK // LANES)) * LANES
            vv = vv_ref[c, pl.ds(o, LANES)]
            for k in range(LANES):
                e = i * LANES + k
                vs = jnp.full((LANES,), vv[k], jnp.float32)
                w = gbuf[e, pl.ds(0, LANES)]
                lo = lax.bitcast_convert_type(w << 16, jnp.float32)
                hi = lax.bitcast_convert_type(w & himask, jnp.float32)
                mbuf[e, pl.ds(0, LANES)] = lo * vs
                mbuf[e, pl.ds(LANES, LANES)] = hi * vs
            return 0
        lax.fori_loop(0, GROUP // LANES, body, 0, unroll=False)

    def run(ego, out):
        for cp in meta_copies(0, 0, semm[0], False):
            cp.wait()
        meta_copies(1, 1, semm[1], False)
        issue_gathers(ego, 0, 0)

        def outer(g4, _):
            for j in range(4):
                i = g4 * 4 + j
                p, q = j % 2, 1 - (j % 2)
                for c in range(GC):
                    pltpu.make_async_copy(
                        ego.at[cols[j].at[c]],
                        gbufs[p].at[pl.ds(c * CHUNK, CHUNK)],
                        semg[p]).wait()
                @pl.when(i + 2 < NGROUPS)
                def _():
                    meta_copies(i + 2, (j + 2) % 4, semm[p], False)
                @pl.when(i + 1 < NGROUPS)
                def _():
                    for cp in meta_copies(i + 1, (j + 1) % 4, semm[q], True):
                        cp.wait()

                @pl.when(i >= 1)
                def _():
                    for c in range(GC):
                        pltpu.make_async_copy(
                            mbufs[q].at[pl.ds(c * CHUNK, CHUNK)],
                            acc.at[rows[(j + 3) % 4].at[c]],
                            sems[q]).wait()

                @pl.when(i + 1 < NGROUPS)
                def _():
                    issue_gathers(ego, (j + 1) % 4, q)

                scale(j, p)
                for c in range(GC):
                    pltpu.async_copy(
                        mbufs[p].at[pl.ds(c * CHUNK, CHUNK)],
                        acc.at[rows[j].at[c]], sems[p], add=True)
            return 0

        lax.fori_loop(0, NGROUPS // 4, outer, 0, unroll=False)

        for c in range(GC):
            pltpu.make_async_copy(
                mbufs[1].at[pl.ds(c * CHUNK, CHUNK)],
                acc.at[rows[3].at[c]], sems[1]).wait()

        plsc.subcore_barrier()
        pltpu.sync_copy(acc.at[pl.ds(sub * ZPT, ZPT)],
                        out.at[pl.ds(sub * ZPT, ZPT)])

    @pl.when(core == 0)
    def _():
        run(ego_lo, out_lo)

    @pl.when(core == 1)
    def _():
        run(ego_hi, out_hi)


@jax.jit
def _spmm(ego_lo, ego_hi, adj3, val2):
    zrows = jnp.zeros((ZPT, DH), jnp.float32)
    mesh = plsc.VectorSubcoreMesh(core_axis_name="c", subcore_axis_name="s")
    half = jax.ShapeDtypeStruct((NP, DH), jnp.float32)
    idxbuf = pltpu.VMEM((GC, CHUNK), jnp.int32)
    valbuf = pltpu.VMEM((GC, CHUNK), jnp.float32)
    return pl.kernel(
        _spmm_body,
        out_type=(half, half),
        mesh=mesh,
        scratch_types=[
            idxbuf, idxbuf, idxbuf, idxbuf,
            idxbuf, idxbuf, idxbuf, idxbuf,
            valbuf, valbuf, valbuf, valbuf,
            pltpu.VMEM((GROUP, LANES), jnp.int32),
            pltpu.VMEM((GROUP, LANES), jnp.int32),
            pltpu.VMEM((GROUP, DH), jnp.float32),
            pltpu.VMEM((GROUP, DH), jnp.float32),
            pltpu.SemaphoreType.DMA,
            pltpu.SemaphoreType.DMA,
            pltpu.SemaphoreType.DMA,
            pltpu.SemaphoreType.DMA,
            pltpu.SemaphoreType.DMA,
            pltpu.SemaphoreType.DMA,
            pltpu.VMEM_SHARED((NP, DH), jnp.float32),
        ],
        compiler_params=pltpu.CompilerParams(use_tc_tiling_on_sc=False),
    )(ego_lo, ego_hi, adj3, val2, zrows)


def _dense_body(elo, ehi, slo, shi, wgl, wgh, bg4, wbl, wbh, bb4, ones4,
                new_lo, new_hi, norm, blo, bhi):
    e_lo, e_hi = elo[...], ehi[...]
    s_lo, s_hi = slo[...], shi[...]
    x = (jnp.dot(s_lo, wgl[...], preferred_element_type=jnp.float32)
         + jnp.dot(s_hi, wgh[...], preferred_element_type=jnp.float32)
         + jnp.dot(e_lo * s_lo, wbl[...], preferred_element_type=jnp.float32)
         + jnp.dot(e_hi * s_hi, wbh[...], preferred_element_type=jnp.float32)
         + bg4[...] + bb4[...])
    h = jnp.where(x >= 0, x, 0.2 * x)
    n2 = jnp.dot(h * h, ones4[...], preferred_element_type=jnp.float32)
    new_lo[...] = jnp.concatenate(
        [h[:, k * D:k * D + DH] for k in range(F)], axis=1)
    new_hi[...] = jnp.concatenate(
        [h[:, k * D + DH:(k + 1) * D] for k in range(F)], axis=1)
    blo[...] = _pack_fold(h, 0)
    bhi[...] = _pack_fold(h, DH)
    norm[...] = h / jnp.maximum(jnp.sqrt(n2), 1e-12)


@jax.jit
def _dense(elo, ehi, slo, shi, wgl, wgh, bg4, wbl, wbh, bb4, ones4):
    blk = pl.BlockSpec((DB, 128), lambda i: (i, 0))
    blk_w = pl.BlockSpec((DB, F * D), lambda i: (i, 0))
    wspec = pl.BlockSpec((128, F * D), lambda i: (0, 0))
    ospec = pl.BlockSpec((F * D, F * D), lambda i: (0, 0))
    bspec = pl.BlockSpec((1, F * D), lambda i: (0, 0))
    pblk = pl.BlockSpec((DB, DH * 2), lambda i: (i, 0))
    fold = jax.ShapeDtypeStruct((NPF, 128), jnp.float32)
    foldp = jax.ShapeDtypeStruct((NPF, DH * 2), jnp.int32)
    return pl.pallas_call(
        _dense_body,
        grid=(TC_GRID,),
        in_specs=[blk, blk, blk, blk,
                  wspec, wspec, bspec, wspec, wspec, bspec, ospec],
        out_specs=[blk, blk, blk_w, pblk, pblk],
        out_shape=[fold, fold,
                   jax.ShapeDtypeStruct((NPF, F * D), jnp.float32),
                   foldp, foldp],
    )(elo, ehi, slo, shi, wgl, wgh, bg4, wbl, wbh, bb4, ones4)


def _pack_body(e0, glo, ghi):
    x = e0[...]
    glo[...] = _pack_fold(x, 0)
    ghi[...] = _pack_fold(x, DH)


@jax.jit
def _pack(e0f):
    pblk = pl.BlockSpec((DB, DH * 2), lambda i: (i, 0))
    foldp = jax.ShapeDtypeStruct((NPF, DH * 2), jnp.int32)
    return pl.pallas_call(
        _pack_body,
        grid=(TC_GRID,),
        in_specs=[pl.BlockSpec((DB, F * D), lambda i: (i, 0))],
        out_specs=[pblk, pblk],
        out_shape=[foldp, foldp],
    )(e0f)


def _lookup_body(s0, s1, s2, s3, idx, o0, o1, o2, o3, idxv, rbuf, sem):
    core = lax.axis_index("c")
    sub = lax.axis_index("s")
    wid = sub * NC + core
    base = wid * GPW
    pltpu.sync_copy(idx.at[pl.ds(base, GPW)], idxv)
    for slab, o in ((s0, o0), (s1, o1), (s2, o2), (s3, o3)):
        pltpu.async_copy(slab.at[idxv], rbuf, sem).wait()
        pltpu.sync_copy(rbuf, o.at[pl.ds(base, GPW)])


@jax.jit
def _lookup(s0, s1, s2, s3, idx):
    mesh = plsc.VectorSubcoreMesh(core_axis_name="c", subcore_axis_name="s")
    out = jax.ShapeDtypeStruct((GB, D), jnp.float32)
    return pl.kernel(
        _lookup_body,
        out_type=(out, out, out, out),
        mesh=mesh,
        scratch_types=[
            pltpu.VMEM((GPW,), jnp.int32),
            pltpu.VMEM((GPW, D), jnp.float32),
            pltpu.SemaphoreType.DMA,
        ],
        compiler_params=pltpu.CompilerParams(use_tc_tiling_on_sc=False),
    )(s0, s1, s2, s3, idx)


def kernel(user_emb, item_emb,
           W_gc_0, b_gc_0, W_bi_0, b_bi_0,
           W_gc_1, b_gc_1, W_bi_1, b_bi_1,
           W_gc_2, b_gc_2, W_bi_2, b_bi_2,
           adj_vals, adj_idx, users, pos_items, neg_items):
    Ws = [(W_gc_0, b_gc_0, W_bi_0, b_bi_0),
          (W_gc_1, b_gc_1, W_bi_1, b_bi_1),
          (W_gc_2, b_gc_2, W_bi_2, b_bi_2)]

    ego0 = jnp.zeros((NP, D), jnp.float32)
    ego0 = ego0.at[:N_USER].set(user_emb).at[N_USER:N].set(item_emb)

    adj3 = jnp.pad(adj_idx.astype(jnp.int32),
                   ((0, 0), (0, NNZ_P - NNZ))).reshape(2, -1, CHUNK)
    val2 = jnp.pad(adj_vals.astype(jnp.float32),
                   (0, NNZ_P - NNZ)).reshape(-1, CHUNK)

    eye4 = jnp.eye(F, dtype=jnp.float32)
    ones4 = jnp.kron(eye4, jnp.ones((D, D), jnp.float32))

    slabs = [ego0]
    glo, ghi = _pack(ego0.reshape(NPF, F * D))
    glo = glo.reshape(NP, LANES)
    ghi = ghi.reshape(NP, LANES)
    ef_lo = ego0[:, :DH].reshape(NPF, 128)
    ef_hi = ego0[:, DH:].reshape(NPF, 128)
    for (wgc, bgc, wbi, bbi) in Ws:
        slo, shi = _spmm(glo, ghi, adj3, val2)
        wgl = jnp.kron(eye4, wgc[:DH])
        wgh = jnp.kron(eye4, wgc[DH:])
        wbl = jnp.kron(eye4, wbi[:DH])
        wbh = jnp.kron(eye4, wbi[DH:])
        bg4 = jnp.tile(bgc, (1, F))
        bb4 = jnp.tile(bbi, (1, F))
        ef_lo, ef_hi, nrm, blo, bhi = _dense(
            ef_lo, ef_hi, slo.reshape(NPF, 128), shi.reshape(NPF, 128),
            wgl, wgh, bg4, wbl, wbh, bb4, ones4)
        glo = blo.reshape(NP, LANES)
        ghi = bhi.reshape(NP, LANES)
        slabs.append(nrm.reshape(NP, D))

    idx = jnp.concatenate([users.astype(jnp.int32),
                           pos_items.astype(jnp.int32) + N_USER,
                           neg_items.astype(jnp.int32) + N_USER])
    g0, g1, g2, g3 = _lookup(slabs[0], slabs[1], slabs[2], slabs[3], idx)
    all_e = jnp.concatenate([g0, g1, g2, g3], axis=1)
    B = users.shape[0]
    return (all_e[:B], all_e[B:2 * B], all_e[2 * B:])


# speedup vs baseline: 6.9988x; 1.2141x over previous
import functools

import jax
import jax.numpy as jnp
from jax import lax
from jax.experimental import pallas as pl
from jax.experimental.pallas import tpu as pltpu
from jax.experimental.pallas import tpu_sc as plsc

N_USER = 25000
N_ITEM = 25000
N = N_USER + N_ITEM
D = 64
DH = D // 2
NNZ = 800000
LANES = 16

NC = 2
NS = 16
NW = NC * NS

NP = 50176
ZPT = NP // NS

NNZ_P = 811008
EPT = NNZ_P // NS
CHUNK = 128
GC = 3
GROUP = CHUNK * GC
NGROUPS = EPT // GROUP

F = 4
NPF = NP // F
DB = 448
TC_GRID = NPF // DB

GB = 3 * 1024
GPW = GB // NW


def _spmm_body(ego_lo, ego_hi, row1, col1, val1, zrows, out_lo, out_hi,
               r00, r01, r02, r10, r11, r12, r20, r21, r22, r30, r31, r32,
               c0, c1, c2, c3, v0, v1, v2, v3, gb0, gb1,
               semg0, semg1, sems0, sems1, semm0, semm1, acc):
    core = lax.axis_index("c")
    sub = lax.axis_index("s")
    rows = [[r00, r01, r02], [r10, r11, r12], [r20, r21, r22], [r30, r31, r32]]
    cols = [c0, c1, c2, c3]
    vals = [v0, v1, v2, v3]
    gbufs = [gb0, gb1]
    semg = [semg0, semg1]
    sems = [sems0, sems1]
    semm = [semm0, semm1]

    pltpu.sync_copy(zrows, acc.at[pl.ds(sub * ZPT, ZPT)])
    plsc.subcore_barrier()

    ebase = sub * EPT

    def meta_copies(i, slot, sem, make):
        f = pltpu.make_async_copy if make else pltpu.async_copy
        eb = ebase + i * GROUP
        cps = [f(col1.at[pl.ds(eb, GROUP)], cols[slot], sem),
               f(val1.at[pl.ds(eb, GROUP)], vals[slot], sem)]
        cps += [f(row1.at[pl.ds(eb + c * CHUNK, CHUNK)], rows[slot][c], sem)
                for c in range(GC)]
        return cps

    def issue_gathers(ego, slot, p):
        return [pltpu.async_copy(
            ego.at[cols[slot].at[pl.ds(c * CHUNK, CHUNK)]],
            gbufs[p].at[pl.ds(c * CHUNK, CHUNK)], semg[p])
            for c in range(GC)]

    def scale(slot, p):
        gbuf = gbufs[p]
        vv_ref = vals[slot]

        def body(i, _):
            vv = vv_ref[pl.ds(i * LANES, LANES)]
            for k in range(LANES):
                e = i * LANES + k
                vs = jnp.full((LANES,), vv[k], jnp.float32)
                for q in range(DH // LANES):
                    sl = pl.ds(q * LANES, LANES)
                    gbuf[e, sl] = gbuf[e, sl] * vs
            return 0
        lax.fori_loop(0, GROUP // LANES, body, 0, unroll=False)

    def run(ego, out):
        for cp in meta_copies(0, 0, semm[0], False):
            cp.wait()
        meta_copies(1, 1, semm[1], False)
        issue_gathers(ego, 0, 0)

        def outer(g4, _):
            for j in range(4):
                i = g4 * 4 + j
                p, q = j % 2, 1 - (j % 2)
                for c in range(GC):
                    pltpu.make_async_copy(
                        ego.at[cols[j].at[pl.ds(c * CHUNK, CHUNK)]],
                        gbufs[p].at[pl.ds(c * CHUNK, CHUNK)],
                        semg[p]).wait()
                @pl.when(i + 2 < NGROUPS)
                def _():
                    meta_copies(i + 2, (j + 2) % 4, semm[p], False)
                @pl.when(i + 1 < NGROUPS)
                def _():
                    for cp in meta_copies(i + 1, (j + 1) % 4, semm[q], True):
                        cp.wait()

                @pl.when(i >= 1)
                def _():
                    for c in range(GC):
                        pltpu.make_async_copy(
                            gbufs[q].at[pl.ds(c * CHUNK, CHUNK)],
                            acc.at[rows[(j + 3) % 4][c]],
                            sems[q]).wait()

                @pl.when(i + 1 < NGROUPS)
                def _():
                    issue_gathers(ego, (j + 1) % 4, q)

                scale(j, p)
                for c in range(GC):
                    pltpu.async_copy(
                        gbufs[p].at[pl.ds(c * CHUNK, CHUNK)],
                        acc.at[rows[j][c]], sems[p], add=True)
            return 0

        lax.fori_loop(0, NGROUPS // 4, outer, 0, unroll=False)

        for c in range(GC):
            pltpu.make_async_copy(
                gbufs[1].at[pl.ds(c * CHUNK, CHUNK)],
                acc.at[rows[3][c]], sems[1]).wait()

        plsc.subcore_barrier()
        pltpu.sync_copy(acc.at[pl.ds(sub * ZPT, ZPT)],
                        out.at[pl.ds(sub * ZPT, ZPT)])

    @pl.when(core == 0)
    def _():
        run(ego_lo, out_lo)

    @pl.when(core == 1)
    def _():
        run(ego_hi, out_hi)


@jax.jit
def _spmm(ego_lo, ego_hi, row1, col1, val1):
    zrows = jnp.zeros((ZPT, DH), jnp.float32)
    mesh = plsc.VectorSubcoreMesh(core_axis_name="c", subcore_axis_name="s")
    half = jax.ShapeDtypeStruct((NP, DH), jnp.float32)
    rbuf = pltpu.VMEM((CHUNK,), jnp.int32)
    cbuf = pltpu.VMEM((GROUP,), jnp.int32)
    valbuf = pltpu.VMEM((GROUP,), jnp.float32)
    return pl.kernel(
        _spmm_body,
        out_type=(half, half),
        mesh=mesh,
        scratch_types=[
            rbuf, rbuf, rbuf, rbuf, rbuf, rbuf,
            rbuf, rbuf, rbuf, rbuf, rbuf, rbuf,
            cbuf, cbuf, cbuf, cbuf,
            valbuf, valbuf, valbuf, valbuf,
            pltpu.VMEM((GROUP, DH), jnp.float32),
            pltpu.VMEM((GROUP, DH), jnp.float32),
            pltpu.SemaphoreType.DMA,
            pltpu.SemaphoreType.DMA,
            pltpu.SemaphoreType.DMA,
            pltpu.SemaphoreType.DMA,
            pltpu.SemaphoreType.DMA,
            pltpu.SemaphoreType.DMA,
            pltpu.VMEM_SHARED((NP, DH), jnp.float32),
        ],
        compiler_params=pltpu.CompilerParams(use_tc_tiling_on_sc=False),
    )(ego_lo, ego_hi, row1, col1, val1, zrows)


def _dense_body(elo, ehi, slo, shi, wgl, wgh, bg4, wbl, wbh, bb4, ones4,
                new_lo, new_hi, norm):
    e_lo, e_hi = elo[...], ehi[...]
    s_lo, s_hi = slo[...], shi[...]
    x = (jnp.dot(s_lo, wgl[...], preferred_element_type=jnp.float32)
         + jnp.dot(s_hi, wgh[...], preferred_element_type=jnp.float32)
         + jnp.dot(e_lo * s_lo, wbl[...], preferred_element_type=jnp.float32)
         + jnp.dot(e_hi * s_hi, wbh[...], preferred_element_type=jnp.float32)
         + bg4[...] + bb4[...])
    h = jnp.where(x >= 0, x, 0.2 * x)
    n2 = jnp.dot(h * h, ones4[...], preferred_element_type=jnp.float32)
    new_lo[...] = jnp.concatenate(
        [h[:, k * D:k * D + DH] for k in range(F)], axis=1)
    new_hi[...] = jnp.concatenate(
        [h[:, k * D + DH:(k + 1) * D] for k in range(F)], axis=1)
    norm[...] = h / jnp.maximum(jnp.sqrt(n2), 1e-12)


@jax.jit
def _dense(elo, ehi, slo, shi, wgl, wgh, bg4, wbl, wbh, bb4, ones4):
    blk = pl.BlockSpec((DB, 128), lambda i: (i, 0))
    blk_w = pl.BlockSpec((DB, F * D), lambda i: (i, 0))
    wspec = pl.BlockSpec((128, F * D), lambda i: (0, 0))
    ospec = pl.BlockSpec((F * D, F * D), lambda i: (0, 0))
    bspec = pl.BlockSpec((1, F * D), lambda i: (0, 0))
    fold = jax.ShapeDtypeStruct((NPF, 128), jnp.float32)
    return pl.pallas_call(
        _dense_body,
        grid=(TC_GRID,),
        in_specs=[blk, blk, blk, blk,
                  wspec, wspec, bspec, wspec, wspec, bspec, ospec],
        out_specs=[blk, blk, blk_w],
        out_shape=[fold, fold,
                   jax.ShapeDtypeStruct((NPF, F * D), jnp.float32)],
    )(elo, ehi, slo, shi, wgl, wgh, bg4, wbl, wbh, bb4, ones4)


def _lookup_body(s0, s1, s2, s3, idx, o0, o1, o2, o3, idxv, rbuf, sem):
    core = lax.axis_index("c")
    sub = lax.axis_index("s")
    wid = sub * NC + core
    base = wid * GPW
    pltpu.sync_copy(idx.at[pl.ds(base, GPW)], idxv)
    for slab, o in ((s0, o0), (s1, o1), (s2, o2), (s3, o3)):
        pltpu.async_copy(slab.at[idxv], rbuf, sem).wait()
        pltpu.sync_copy(rbuf, o.at[pl.ds(base, GPW)])


@jax.jit
def _lookup(s0, s1, s2, s3, idx):
    mesh = plsc.VectorSubcoreMesh(core_axis_name="c", subcore_axis_name="s")
    out = jax.ShapeDtypeStruct((GB, D), jnp.float32)
    return pl.kernel(
        _lookup_body,
        out_type=(out, out, out, out),
        mesh=mesh,
        scratch_types=[
            pltpu.VMEM((GPW,), jnp.int32),
            pltpu.VMEM((GPW, D), jnp.float32),
            pltpu.SemaphoreType.DMA,
        ],
        compiler_params=pltpu.CompilerParams(use_tc_tiling_on_sc=False),
    )(s0, s1, s2, s3, idx)


def kernel(user_emb, item_emb,
           W_gc_0, b_gc_0, W_bi_0, b_bi_0,
           W_gc_1, b_gc_1, W_bi_1, b_bi_1,
           W_gc_2, b_gc_2, W_bi_2, b_bi_2,
           adj_vals, adj_idx, users, pos_items, neg_items):
    Ws = [(W_gc_0, b_gc_0, W_bi_0, b_bi_0),
          (W_gc_1, b_gc_1, W_bi_1, b_bi_1),
          (W_gc_2, b_gc_2, W_bi_2, b_bi_2)]

    ego0 = jnp.zeros((NP, D), jnp.float32)
    ego0 = ego0.at[:N_USER].set(user_emb).at[N_USER:N].set(item_emb)

    row1 = jnp.pad(adj_idx[0].astype(jnp.int32), (0, NNZ_P - NNZ))
    col1 = jnp.pad(adj_idx[1].astype(jnp.int32), (0, NNZ_P - NNZ))
    val1 = jnp.pad(adj_vals.astype(jnp.float32), (0, NNZ_P - NNZ))

    eye4 = jnp.eye(F, dtype=jnp.float32)
    ones4 = jnp.kron(eye4, jnp.ones((D, D), jnp.float32))

    slabs = [ego0]
    elo, ehi = ego0[:, :DH], ego0[:, DH:]
    for (wgc, bgc, wbi, bbi) in Ws:
        slo, shi = _spmm(elo, ehi, row1, col1, val1)
        wgl = jnp.kron(eye4, wgc[:DH])
        wgh = jnp.kron(eye4, wgc[DH:])
        wbl = jnp.kron(eye4, wbi[:DH])
        wbh = jnp.kron(eye4, wbi[DH:])
        bg4 = jnp.tile(bgc, (1, F))
        bb4 = jnp.tile(bbi, (1, F))
        nlo, nhi, nrm = _dense(elo.reshape(NPF, 128), ehi.reshape(NPF, 128),
                               slo.reshape(NPF, 128), shi.reshape(NPF, 128),
                               wgl, wgh, bg4, wbl, wbh, bb4, ones4)
        elo = nlo.reshape(NP, DH)
        ehi = nhi.reshape(NP, DH)
        slabs.append(nrm.reshape(NP, D))

    idx = jnp.concatenate([users.astype(jnp.int32),
                           pos_items.astype(jnp.int32) + N_USER,
                           neg_items.astype(jnp.int32) + N_USER])
    g0, g1, g2, g3 = _lookup(slabs[0], slabs[1], slabs[2], slabs[3], idx)
    all_e = jnp.concatenate([g0, g1, g2, g3], axis=1)
    B = users.shape[0]
    return (all_e[:B], all_e[B:2 * B], all_e[2 * B:])


# speedup vs baseline: 7.3991x; 1.0572x over previous
import functools

import jax
import jax.numpy as jnp
from jax import lax
from jax.experimental import pallas as pl
from jax.experimental.pallas import tpu as pltpu
from jax.experimental.pallas import tpu_sc as plsc

N_USER = 25000
N_ITEM = 25000
N = N_USER + N_ITEM
D = 64
DH = D // 2
NNZ = 800000
LANES = 16

NC = 2
NS = 16
NW = NC * NS

NP = 50176
ZPT = NP // NS

NNZ_P = 811008
EPT = NNZ_P // NS
CHUNK = 128
GC = 3
GROUP = CHUNK * GC
NGROUPS = EPT // GROUP

F = 4
NPF = NP // F
DB = 448
TC_GRID = NPF // DB

GB = 3 * 1024
GPW = GB // NW


def _spmm_body(ego_lo, ego_hi, adj3, val2, zrows, out_lo, out_hi,
               r0, r1, r2, r3, c0, c1, c2, c3, v0, v1, v2, v3, gb0, gb1,
               semg0, semg1, sems0, sems1, semm0, semm1, acc):
    core = lax.axis_index("c")
    sub = lax.axis_index("s")
    rows = [r0, r1, r2, r3]
    cols = [c0, c1, c2, c3]
    vals = [v0, v1, v2, v3]
    gbufs = [gb0, gb1]
    semg = [semg0, semg1]
    sems = [sems0, sems1]
    semm = [semm0, semm1]

    pltpu.sync_copy(zrows, acc.at[pl.ds(sub * ZPT, ZPT)])
    plsc.subcore_barrier()

    cbase = sub * (EPT // CHUNK)

    def meta_copies(i, slot, sem, make):
        f = pltpu.make_async_copy if make else pltpu.async_copy
        sl = pl.ds(cbase + i * GC, GC)
        return [f(adj3.at[0, sl], rows[slot], sem),
                f(adj3.at[1, sl], cols[slot], sem),
                f(val2.at[sl], vals[slot], sem)]

    def issue_gathers(ego, slot, p):
        return [pltpu.async_copy(
            ego.at[cols[slot].at[c]],
            gbufs[p].at[pl.ds(c * CHUNK, CHUNK)], semg[p])
            for c in range(GC)]

    def scale(slot, p):
        gbuf = gbufs[p]
        vv_ref = vals[slot]

        def body(i, _):
            c = i // (CHUNK // LANES)
            o = (i % (CHUNK // LANES)) * LANES
            vv = vv_ref[c, pl.ds(o, LANES)]
            for k in range(LANES):
                e = i * LANES + k
                vs = jnp.full((LANES,), vv[k], jnp.float32)
                for q in range(DH // LANES):
                    sl = pl.ds(q * LANES, LANES)
                    gbuf[e, sl] = gbuf[e, sl] * vs
            return 0
        lax.fori_loop(0, GROUP // LANES, body, 0, unroll=False)

    def run(ego, out):
        for cp in meta_copies(0, 0, semm[0], False):
            cp.wait()
        meta_copies(1, 1, semm[1], False)
        issue_gathers(ego, 0, 0)

        def outer(g4, _):
            for j in range(4):
                i = g4 * 4 + j
                p, q = j % 2, 1 - (j % 2)
                for c in range(GC):
                    pltpu.make_async_copy(
                        ego.at[cols[j].at[c]],
                        gbufs[p].at[pl.ds(c * CHUNK, CHUNK)],
                        semg[p]).wait()
                @pl.when(i + 2 < NGROUPS)
                def _():
                    meta_copies(i + 2, (j + 2) % 4, semm[p], False)
                @pl.when(i + 1 < NGROUPS)
                def _():
                    for cp in meta_copies(i + 1, (j + 1) % 4, semm[q], True):
                        cp.wait()

                @pl.when(i >= 1)
                def _():
                    for c in range(GC):
                        pltpu.make_async_copy(
                            gbufs[q].at[pl.ds(c * CHUNK, CHUNK)],
                            acc.at[rows[(j + 3) % 4].at[c]],
                            sems[q]).wait()

                @pl.when(i + 1 < NGROUPS)
                def _():
                    issue_gathers(ego, (j + 1) % 4, q)

                scale(j, p)
                for c in range(GC):
                    pltpu.async_copy(
                        gbufs[p].at[pl.ds(c * CHUNK, CHUNK)],
                        acc.at[rows[j].at[c]], sems[p], add=True)
            return 0

        lax.fori_loop(0, NGROUPS // 4, outer, 0, unroll=False)

        for c in range(GC):
            pltpu.make_async_copy(
                gbufs[1].at[pl.ds(c * CHUNK, CHUNK)],
                acc.at[rows[3].at[c]], sems[1]).wait()

        plsc.subcore_barrier()
        pltpu.sync_copy(acc.at[pl.ds(sub * ZPT, ZPT)],
                        out.at[pl.ds(sub * ZPT, ZPT)])

    @pl.when(core == 0)
    def _():
        run(ego_lo, out_lo)

    @pl.when(core == 1)
    def _():
        run(ego_hi, out_hi)


@jax.jit
def _spmm(ego_lo, ego_hi, adj3, val2):
    zrows = jnp.zeros((ZPT, DH), jnp.float32)
    mesh = plsc.VectorSubcoreMesh(core_axis_name="c", subcore_axis_name="s")
    half = jax.ShapeDtypeStruct((NP, DH), jnp.float32)
    idxbuf = pltpu.VMEM((GC, CHUNK), jnp.int32)
    valbuf = pltpu.VMEM((GC, CHUNK), jnp.float32)
    return pl.kernel(
        _spmm_body,
        out_type=(half, half),
        mesh=mesh,
        scratch_types=[
            idxbuf, idxbuf, idxbuf, idxbuf,
            idxbuf, idxbuf, idxbuf, idxbuf,
            valbuf, valbuf, valbuf, valbuf,
            pltpu.VMEM((GROUP, DH), jnp.float32),
            pltpu.VMEM((GROUP, DH), jnp.float32),
            pltpu.SemaphoreType.DMA,
            pltpu.SemaphoreType.DMA,
            pltpu.SemaphoreType.DMA,
            pltpu.SemaphoreType.DMA,
            pltpu.SemaphoreType.DMA,
            pltpu.SemaphoreType.DMA,
            pltpu.VMEM_SHARED((NP, DH), jnp.float32),
        ],
        compiler_params=pltpu.CompilerParams(use_tc_tiling_on_sc=False),
    )(ego_lo, ego_hi, adj3, val2, zrows)


def _dense_body(elo, ehi, slo, shi, wgl, wgh, bg4, wbl, wbh, bb4, ones4,
                new_lo, new_hi, norm):
    e_lo, e_hi = elo[...], ehi[...]
    s_lo, s_hi = slo[...], shi[...]
    x = (jnp.dot(s_lo, wgl[...], preferred_element_type=jnp.float32)
         + jnp.dot(s_hi, wgh[...], preferred_element_type=jnp.float32)
         + jnp.dot(e_lo * s_lo, wbl[...], preferred_element_type=jnp.float32)
         + jnp.dot(e_hi * s_hi, wbh[...], preferred_element_type=jnp.float32)
         + bg4[...] + bb4[...])
    h = jnp.where(x >= 0, x, 0.2 * x)
    n2 = jnp.dot(h * h, ones4[...], preferred_element_type=jnp.float32)
    new_lo[...] = jnp.concatenate(
        [h[:, k * D:k * D + DH] for k in range(F)], axis=1)
    new_hi[...] = jnp.concatenate(
        [h[:, k * D + DH:(k + 1) * D] for k in range(F)], axis=1)
    norm[...] = h / jnp.maximum(jnp.sqrt(n2), 1e-12)


@jax.jit
def _dense(elo, ehi, slo, shi, wgl, wgh, bg4, wbl, wbh, bb4, ones4):
    blk = pl.BlockSpec((DB, 128), lambda i: (i, 0))
    blk_w = pl.BlockSpec((DB, F * D), lambda i: (i, 0))
    wspec = pl.BlockSpec((128, F * D), lambda i: (0, 0))
    ospec = pl.BlockSpec((F * D, F * D), lambda i: (0, 0))
    bspec = pl.BlockSpec((1, F * D), lambda i: (0, 0))
    fold = jax.ShapeDtypeStruct((NPF, 128), jnp.float32)
    return pl.pallas_call(
        _dense_body,
        grid=(TC_GRID,),
        in_specs=[blk, blk, blk, blk,
                  wspec, wspec, bspec, wspec, wspec, bspec, ospec],
        out_specs=[blk, blk, blk_w],
        out_shape=[fold, fold,
                   jax.ShapeDtypeStruct((NPF, F * D), jnp.float32)],
    )(elo, ehi, slo, shi, wgl, wgh, bg4, wbl, wbh, bb4, ones4)


def _lookup_body(s0, s1, s2, s3, idx, o0, o1, o2, o3, idxv, rbuf, sem):
    core = lax.axis_index("c")
    sub = lax.axis_index("s")
    wid = sub * NC + core
    base = wid * GPW
    pltpu.sync_copy(idx.at[pl.ds(base, GPW)], idxv)
    for slab, o in ((s0, o0), (s1, o1), (s2, o2), (s3, o3)):
        pltpu.async_copy(slab.at[idxv], rbuf, sem).wait()
        pltpu.sync_copy(rbuf, o.at[pl.ds(base, GPW)])


@jax.jit
def _lookup(s0, s1, s2, s3, idx):
    mesh = plsc.VectorSubcoreMesh(core_axis_name="c", subcore_axis_name="s")
    out = jax.ShapeDtypeStruct((GB, D), jnp.float32)
    return pl.kernel(
        _lookup_body,
        out_type=(out, out, out, out),
        mesh=mesh,
        scratch_types=[
            pltpu.VMEM((GPW,), jnp.int32),
            pltpu.VMEM((GPW, D), jnp.float32),
            pltpu.SemaphoreType.DMA,
        ],
        compiler_params=pltpu.CompilerParams(use_tc_tiling_on_sc=False),
    )(s0, s1, s2, s3, idx)


def kernel(user_emb, item_emb,
           W_gc_0, b_gc_0, W_bi_0, b_bi_0,
           W_gc_1, b_gc_1, W_bi_1, b_bi_1,
           W_gc_2, b_gc_2, W_bi_2, b_bi_2,
           adj_vals, adj_idx, users, pos_items, neg_items):
    Ws = [(W_gc_0, b_gc_0, W_bi_0, b_bi_0),
          (W_gc_1, b_gc_1, W_bi_1, b_bi_1),
          (W_gc_2, b_gc_2, W_bi_2, b_bi_2)]

    ego0 = jnp.zeros((NP, D), jnp.float32)
    ego0 = ego0.at[:N_USER].set(user_emb).at[N_USER:N].set(item_emb)

    adj3 = jnp.pad(adj_idx.astype(jnp.int32),
                   ((0, 0), (0, NNZ_P - NNZ))).reshape(2, -1, CHUNK)
    val2 = jnp.pad(adj_vals.astype(jnp.float32),
                   (0, NNZ_P - NNZ)).reshape(-1, CHUNK)

    eye4 = jnp.eye(F, dtype=jnp.float32)
    ones4 = jnp.kron(eye4, jnp.ones((D, D), jnp.float32))

    slabs = [ego0]
    elo, ehi = ego0[:, :DH], ego0[:, DH:]
    for (wgc, bgc, wbi, bbi) in Ws:
        slo, shi = _spmm(elo, ehi, adj3, val2)
        wgl = jnp.kron(eye4, wgc[:DH])
        wgh = jnp.kron(eye4, wgc[DH:])
        wbl = jnp.kron(eye4, wbi[:DH])
        wbh = jnp.kron(eye4, wbi[DH:])
        bg4 = jnp.tile(bgc, (1, F))
        bb4 = jnp.tile(bbi, (1, F))
        nlo, nhi, nrm = _dense(elo.reshape(NPF, 128), ehi.reshape(NPF, 128),
                               slo.reshape(NPF, 128), shi.reshape(NPF, 128),
                               wgl, wgh, bg4, wbl, wbh, bb4, ones4)
        elo = nlo.reshape(NP, DH)
        ehi = nhi.reshape(NP, DH)
        slabs.append(nrm.reshape(NP, D))

    idx = jnp.concatenate([users.astype(jnp.int32),
                           pos_items.astype(jnp.int32) + N_USER,
                           neg_items.astype(jnp.int32) + N_USER])
    g0, g1, g2, g3 = _lookup(slabs[0], slabs[1], slabs[2], slabs[3], idx)
    all_e = jnp.concatenate([g0, g1, g2, g3], axis=1)
    B = users.shape[0]
    return (all_e[:B], all_e[B:2 * B], all_e[2 * B:])
